# Initial kernel scaffold; baseline (speedup 1.0000x reference)
import jax
import jax.numpy as jnp
from jax.experimental import pallas as pl


def kernel(edge_index, emb_table, W_self_0, W_neigh_0, b_0, W_self_1, W_neigh_1, b_1, W_self_2, W_neigh_2, b_2):
    raise NotImplementedError("write your pallas kernel here")


# speedup vs baseline: 3.0156x; 3.0156x over previous
import functools

import jax
import jax.numpy as jnp
from jax import lax
from jax.experimental import pallas as pl
from jax.experimental.pallas import tpu as pltpu
from jax.experimental.pallas import tpu_sc as plsc

_N = 10000
_D = 128
_E = 320000
_NC = 2
_NS = 16
_NW = _NC * _NS
_CH = 128
_C = 80
_EP = _NW * _C * _CH
_NP = 10240
_RT = _NP // _NS

_mesh = plsc.VectorSubcoreMesh(core_axis_name="c", subcore_axis_name="s",
                               num_cores=_NC, num_subcores=_NS)


def _sc_agg_body(w, x_hbm, idx_hbm, zeros_hbm, out_hbm,
                 idx_v, srcc, rows_v, sem, sem2, acc_sh):
    cid = lax.axis_index("c")
    sid = lax.axis_index("s")
    wid = cid * _NS + sid
    r0 = sid * _RT

    pltpu.sync_copy(zeros_hbm.at[pl.ds(0, _CH)], rows_v)
    for c in range(_RT // _CH):
        pltpu.sync_copy(rows_v, acc_sh.at[pl.ds(r0 + c * _CH, _CH)])

    pltpu.sync_copy(idx_hbm.at[wid], idx_v)
    plsc.subcore_barrier()

    def step(j, _):
        for k in range(_CH // 16):
            pk = idx_v[j, pl.ds(k * 16, 16)]
            srcc[pl.ds(k * 16, 16)] = lax.bitwise_and(pk, 16383)
        pltpu.async_copy(x_hbm.at[srcc], rows_v, sem).wait()
        for k in range(_CH // 16):
            pk = idx_v[j, pl.ds(k * 16, 16)]
            dstk = lax.shift_right_logical(pk, 14)
            pltpu.make_async_copy(rows_v.at[pl.ds(k * 16, 16)],
                                  acc_sh.at[dstk], sem2).start(add=True)
        pltpu.make_async_copy(zeros_hbm.at[pl.ds(0, _CH)], rows_v,
                              sem2).wait()
        return 0
    lax.fori_loop(0, _C, step, 0)

    plsc.subcore_barrier()

    for c in range(_RT // _CH):
        pltpu.sync_copy(acc_sh.at[pl.ds(r0 + c * _CH, _CH)], rows_v)
        pltpu.sync_copy(rows_v, out_hbm.at[cid, pl.ds(r0 + c * _CH, _CH)])


def _make_sc_agg(w):
    return pl.kernel(
        functools.partial(_sc_agg_body, w),
        out_type=jax.ShapeDtypeStruct((_NC, _NP, w), jnp.float32),
        mesh=_mesh,
        scratch_types=(
            pltpu.VMEM((_C, _CH), jnp.int32),
            pltpu.VMEM((_CH,), jnp.int32),
            pltpu.VMEM((_CH, w), jnp.float32),
            pltpu.SemaphoreType.DMA,
            pltpu.SemaphoreType.DMA,
            pltpu.VMEM_SHARED((_NP, w), jnp.float32),
        ),
    )


_sc_agg = _make_sc_agg(_D)


def _sc_deg_body(idx_hbm, zeros_hbm, out_hbm, idx_v, rows_v, sem2, acc_sh):
    cid = lax.axis_index("c")
    sid = lax.axis_index("s")
    wid = cid * _NS + sid
    r0 = sid * _RT

    pltpu.sync_copy(zeros_hbm.at[pl.ds(0, _CH)], rows_v)
    for c in range(_RT // _CH):
        pltpu.sync_copy(rows_v, acc_sh.at[pl.ds(r0 + c * _CH, _CH)])
    pltpu.sync_copy(idx_hbm.at[wid], idx_v)

    def fill(i, _):
        for k in range(_D // 16):
            rows_v[i, pl.ds(k * 16, 16)] = jnp.full((16,), 1.0, jnp.float32)
        return 0
    lax.fori_loop(0, _CH, fill, 0)
    plsc.subcore_barrier()

    def step(j, _):
        for k in range(_CH // 16):
            pk = idx_v[j, pl.ds(k * 16, 16)]
            dstk = lax.shift_right_logical(pk, 14)
            pltpu.make_async_copy(rows_v.at[pl.ds(k * 16, 16)],
                                  acc_sh.at[dstk], sem2).start(add=True)
        pltpu.make_async_copy(zeros_hbm.at[pl.ds(0, _CH)], rows_v,
                              sem2).wait()
        return 0
    lax.fori_loop(0, _C, step, 0)

    plsc.subcore_barrier()
    for c in range(_RT // _CH):
        pltpu.sync_copy(acc_sh.at[pl.ds(r0 + c * _CH, _CH)], rows_v)
        pltpu.sync_copy(rows_v, out_hbm.at[cid, pl.ds(r0 + c * _CH, _CH)])


_sc_deg = pl.kernel(
    _sc_deg_body,
    out_type=jax.ShapeDtypeStruct((_NC, _NP, _D), jnp.float32),
    mesh=_mesh,
    scratch_types=(
        pltpu.VMEM((_C, _CH), jnp.int32),
        pltpu.VMEM((_CH, _D), jnp.float32),
        pltpu.SemaphoreType.DMA,
        pltpu.VMEM_SHARED((_NP, _D), jnp.float32),
    ),
)


def _tc_body(act, x_ref, p0_ref, p1_ref, dg_ref, ws_ref, wn_ref, b_ref, o_ref):
    s = p0_ref[...] + p1_ref[...]
    deg = dg_ref[0, :, 0:1] + dg_ref[1, :, 0:1]
    scale = 1.0 / jnp.maximum(deg, 1.0)
    o = (jnp.dot(x_ref[...], ws_ref[...], preferred_element_type=jnp.float32)
         + jnp.dot(s, wn_ref[...], preferred_element_type=jnp.float32) * scale
         + b_ref[...])
    o_ref[...] = jnp.maximum(o, 0.0) if act else o


def _tc_layer(x, p0, p1, degp, ws, wn, b2, act):
    r = 1024
    return pl.pallas_call(
        functools.partial(_tc_body, act),
        grid=(_NP // r,),
        in_specs=[
            pl.BlockSpec((r, _D), lambda i: (i, 0)),
            pl.BlockSpec((r, _D), lambda i: (i, 0)),
            pl.BlockSpec((r, _D), lambda i: (i, 0)),
            pl.BlockSpec((2, r, 16), lambda i: (0, i, 0)),
            pl.BlockSpec((_D, _D), lambda i: (0, 0)),
            pl.BlockSpec((_D, _D), lambda i: (0, 0)),
            pl.BlockSpec((1, _D), lambda i: (0, 0)),
        ],
        out_specs=pl.BlockSpec((r, _D), lambda i: (i, 0)),
        out_shape=jax.ShapeDtypeStruct((_NP, _D), jnp.float32),
    )(x, p0, p1, degp, ws, wn, b2)


def kernel(edge_index, emb_table, W_self_0, W_neigh_0, b_0,
           W_self_1, W_neigh_1, b_1, W_self_2, W_neigh_2, b_2):
    src = edge_index[0]
    dst = edge_index[1]
    pad = _EP - _E
    pad_src = jnp.zeros((pad,), jnp.int32)
    pad_dst = _N + 16 + (jnp.arange(pad, dtype=jnp.int32) % (_NP - _N - 16))
    srcp = jnp.concatenate([src, pad_src])
    dstp = jnp.concatenate([dst, pad_dst])
    packed = jnp.bitwise_or(srcp, jnp.left_shift(dstp, 14)
                            ).reshape(_NW, _C, _CH)

    x = jnp.zeros((_NP, _D), jnp.float32).at[:_N].set(emb_table)
    zeros_d = jnp.zeros((_NP, _D), jnp.float32)

    ws = (W_self_0, W_self_1, W_self_2)
    wn = (W_neigh_0, W_neigh_1, W_neigh_2)
    bs = (b_0, b_1, b_2)

    degp = _sc_deg(packed, zeros_d)[:, :, :16]
    for l in range(3):
        p = _sc_agg(x, packed, zeros_d)
        x = _tc_layer(x, p[0], p[1], degp, ws[l], wn[l],
                      bs[l].reshape(1, _D), act=(l < 2))
    return x[:_N]


# speedup vs baseline: 3.2364x; 1.0732x over previous
import functools

import jax
import jax.numpy as jnp
from jax import lax
from jax.experimental import pallas as pl
from jax.experimental.pallas import tpu as pltpu
from jax.experimental.pallas import tpu_sc as plsc

_N = 10000
_D = 128
_E = 320000
_NC = 2
_NS = 16
_NW = _NC * _NS
_CH = 128
_C = 80
_EP = _NW * _C * _CH
_NP = 10240
_RT = _NP // _NS

_mesh = plsc.VectorSubcoreMesh(core_axis_name="c", subcore_axis_name="s",
                               num_cores=_NC, num_subcores=_NS)


def _sc_agg_body(w, x_hbm, idx_hbm, zeros_hbm, out_hbm,
                 idx_v, src_a, src_b, rows_a, rows_b,
                 sem_a, sem_b, sem_a2, sem_b2, acc_sh):
    cid = lax.axis_index("c")
    sid = lax.axis_index("s")
    wid = cid * _NS + sid
    r0 = sid * _RT

    def unpack(src_ref, j):
        for k in range(_CH // 16):
            pk = idx_v[j, pl.ds(k * 16, 16)]
            src_ref[pl.ds(k * 16, 16)] = lax.bitwise_and(pk, 16383)

    def fire_adds(rows_ref, j, sem2):
        for k in range(_CH // 16):
            pk = idx_v[j, pl.ds(k * 16, 16)]
            dstk = lax.shift_right_logical(pk, 14)
            pltpu.make_async_copy(rows_ref.at[pl.ds(k * 16, 16)],
                                  acc_sh.at[dstk], sem2).start(add=True)

    def drain_adds(rows_ref, sem2):
        pltpu.make_async_copy(zeros_hbm.at[pl.ds(0, _CH)], rows_ref,
                              sem2).wait()

    pltpu.sync_copy(zeros_hbm.at[pl.ds(0, _CH)], rows_a)
    for c in range(_RT // _CH):
        pltpu.sync_copy(rows_a, acc_sh.at[pl.ds(r0 + c * _CH, _CH)])

    pltpu.sync_copy(idx_hbm.at[wid], idx_v)
    plsc.subcore_barrier()

    unpack(src_a, 0)
    pltpu.make_async_copy(x_hbm.at[src_a], rows_a, sem_a).start()

    def step2(j2, _):
        j = 2 * j2
        jn = j + 1
        jnn = j + 2
        pltpu.make_async_copy(x_hbm.at[src_a], rows_a, sem_a).wait()
        fire_adds(rows_a, j, sem_a2)
        unpack(src_b, jn)

        @pl.when(j2 > 0)
        def _():
            drain_adds(rows_b, sem_b2)
        pltpu.make_async_copy(x_hbm.at[src_b], rows_b, sem_b).start()
        pltpu.make_async_copy(x_hbm.at[src_b], rows_b, sem_b).wait()
        fire_adds(rows_b, jn, sem_b2)
        drain_adds(rows_a, sem_a2)

        @pl.when(jnn < _C)
        def _():
            unpack(src_a, jnn)
            pltpu.make_async_copy(x_hbm.at[src_a], rows_a, sem_a).start()
        return 0
    lax.fori_loop(0, _C // 2, step2, 0)
    drain_adds(rows_b, sem_b2)

    plsc.subcore_barrier()

    for c in range(_RT // _CH):
        pltpu.sync_copy(acc_sh.at[pl.ds(r0 + c * _CH, _CH)], rows_a)
        pltpu.sync_copy(rows_a, out_hbm.at[cid, pl.ds(r0 + c * _CH, _CH)])


def _make_sc_agg(w):
    return pl.kernel(
        functools.partial(_sc_agg_body, w),
        out_type=jax.ShapeDtypeStruct((_NC, _NP, w), jnp.float32),
        mesh=_mesh,
        scratch_types=(
            pltpu.VMEM((_C, _CH), jnp.int32),
            pltpu.VMEM((_CH,), jnp.int32),
            pltpu.VMEM((_CH,), jnp.int32),
            pltpu.VMEM((_CH, w), jnp.float32),
            pltpu.VMEM((_CH, w), jnp.float32),
            pltpu.SemaphoreType.DMA,
            pltpu.SemaphoreType.DMA,
            pltpu.SemaphoreType.DMA,
            pltpu.SemaphoreType.DMA,
            pltpu.VMEM_SHARED((_NP, w), jnp.float32),
        ),
    )


_sc_agg = _make_sc_agg(_D)


def _sc_deg_body(idx_hbm, zeros_hbm, out_hbm, idx_v, rows_v, sem2, acc_sh):
    cid = lax.axis_index("c")
    sid = lax.axis_index("s")
    wid = cid * _NS + sid
    r0 = sid * _RT

    pltpu.sync_copy(zeros_hbm.at[pl.ds(0, _CH)], rows_v)
    for c in range(_RT // _CH):
        pltpu.sync_copy(rows_v, acc_sh.at[pl.ds(r0 + c * _CH, _CH)])
    pltpu.sync_copy(idx_hbm.at[wid], idx_v)

    def fill(i, _):
        for k in range(_D // 16):
            rows_v[i, pl.ds(k * 16, 16)] = jnp.full((16,), 1.0, jnp.float32)
        return 0
    lax.fori_loop(0, _CH, fill, 0)
    plsc.subcore_barrier()

    def step(j, _):
        for k in range(_CH // 16):
            pk = idx_v[j, pl.ds(k * 16, 16)]
            dstk = lax.shift_right_logical(pk, 14)
            pltpu.make_async_copy(rows_v.at[pl.ds(k * 16, 16)],
                                  acc_sh.at[dstk], sem2).start(add=True)
        pltpu.make_async_copy(zeros_hbm.at[pl.ds(0, _CH)], rows_v,
                              sem2).wait()
        return 0
    lax.fori_loop(0, _C, step, 0)

    plsc.subcore_barrier()
    for c in range(_RT // _CH):
        pltpu.sync_copy(acc_sh.at[pl.ds(r0 + c * _CH, _CH)], rows_v)
        pltpu.sync_copy(rows_v, out_hbm.at[cid, pl.ds(r0 + c * _CH, _CH)])


_sc_deg = pl.kernel(
    _sc_deg_body,
    out_type=jax.ShapeDtypeStruct((_NC, _NP, _D), jnp.float32),
    mesh=_mesh,
    scratch_types=(
        pltpu.VMEM((_C, _CH), jnp.int32),
        pltpu.VMEM((_CH, _D), jnp.float32),
        pltpu.SemaphoreType.DMA,
        pltpu.VMEM_SHARED((_NP, _D), jnp.float32),
    ),
)


def _tc_body(act, x_ref, p0_ref, p1_ref, dg_ref, ws_ref, wn_ref, b_ref, o_ref):
    s = p0_ref[...] + p1_ref[...]
    deg = dg_ref[0, :, 0:1] + dg_ref[1, :, 0:1]
    scale = 1.0 / jnp.maximum(deg, 1.0)
    o = (jnp.dot(x_ref[...], ws_ref[...], preferred_element_type=jnp.float32)
         + jnp.dot(s, wn_ref[...], preferred_element_type=jnp.float32) * scale
         + b_ref[...])
    o_ref[...] = jnp.maximum(o, 0.0) if act else o


def _tc_layer(x, p0, p1, degp, ws, wn, b2, act):
    r = 1024
    return pl.pallas_call(
        functools.partial(_tc_body, act),
        grid=(_NP // r,),
        in_specs=[
            pl.BlockSpec((r, _D), lambda i: (i, 0)),
            pl.BlockSpec((r, _D), lambda i: (i, 0)),
            pl.BlockSpec((r, _D), lambda i: (i, 0)),
            pl.BlockSpec((2, r, 16), lambda i: (0, i, 0)),
            pl.BlockSpec((_D, _D), lambda i: (0, 0)),
            pl.BlockSpec((_D, _D), lambda i: (0, 0)),
            pl.BlockSpec((1, _D), lambda i: (0, 0)),
        ],
        out_specs=pl.BlockSpec((r, _D), lambda i: (i, 0)),
        out_shape=jax.ShapeDtypeStruct((_NP, _D), jnp.float32),
    )(x, p0, p1, degp, ws, wn, b2)


def kernel(edge_index, emb_table, W_self_0, W_neigh_0, b_0,
           W_self_1, W_neigh_1, b_1, W_self_2, W_neigh_2, b_2):
    src = edge_index[0]
    dst = edge_index[1]
    pad = _EP - _E
    pad_src = jnp.zeros((pad,), jnp.int32)
    pad_dst = _N + 16 + (jnp.arange(pad, dtype=jnp.int32) % (_NP - _N - 16))
    srcp = jnp.concatenate([src, pad_src])
    dstp = jnp.concatenate([dst, pad_dst])
    packed = jnp.bitwise_or(srcp, jnp.left_shift(dstp, 14)
                            ).reshape(_NW, _C, _CH)

    x = jnp.zeros((_NP, _D), jnp.float32).at[:_N].set(emb_table)
    zeros_d = jnp.zeros((_NP, _D), jnp.float32)

    ws = (W_self_0, W_self_1, W_self_2)
    wn = (W_neigh_0, W_neigh_1, W_neigh_2)
    bs = (b_0, b_1, b_2)

    degp = _sc_deg(packed, zeros_d)[:, :, :16]
    for l in range(3):
        p = _sc_agg(x, packed, zeros_d)
        x = _tc_layer(x, p[0], p[1], degp, ws[l], wn[l],
                      bs[l].reshape(1, _D), act=(l < 2))
    return x[:_N]


# speedup vs baseline: 3.5776x; 1.1054x over previous
import functools

import jax
import jax.numpy as jnp
from jax import lax
from jax.experimental import pallas as pl
from jax.experimental.pallas import tpu as pltpu
from jax.experimental.pallas import tpu_sc as plsc

_N = 10000
_D = 128
_E = 320000
_NC = 2
_NS = 16
_NW = _NC * _NS
_CH = 128
_C = 80
_CA = 120
_CB = 40
_EP = _NW * _C * _CH
_NP = 10240
_RT = _NP // _NS

_mesh = plsc.VectorSubcoreMesh(core_axis_name="c", subcore_axis_name="s",
                               num_cores=_NC, num_subcores=_NS)


def _sc_agg_body(w, x_hbm, idx_hbm, zeros_hbm, out_hbm,
                 idx_v, src_a, src_b, rows_a, rows_b,
                 sem_a, sem_b, sem_a2, sem_b2, acc_sh):
    cid = lax.axis_index("c")
    sid = lax.axis_index("s")
    wid = cid * _NS + sid
    r0 = sid * _RT

    def unpack(src_ref, j):
        for k in range(_CH // 16):
            pk = idx_v[j, pl.ds(k * 16, 16)]
            src_ref[pl.ds(k * 16, 16)] = lax.bitwise_and(pk, 16383)

    def fire_adds(rows_ref, j, sem2):
        for k in range(_CH // 16):
            pk = idx_v[j, pl.ds(k * 16, 16)]
            dstk = lax.shift_right_logical(pk, 14)
            pltpu.make_async_copy(rows_ref.at[pl.ds(k * 16, 16)],
                                  acc_sh.at[dstk], sem2).start(add=True)

    def drain_adds(rows_ref, sem2):
        pltpu.make_async_copy(zeros_hbm.at[pl.ds(0, _CH)], rows_ref,
                              sem2).wait()

    pltpu.sync_copy(zeros_hbm.at[pl.ds(0, _CH)], rows_a)
    for c in range(_RT // _CH):
        pltpu.sync_copy(rows_a, acc_sh.at[pl.ds(r0 + c * _CH, _CH)])

    pltpu.sync_copy(idx_hbm.at[wid], idx_v)
    plsc.subcore_barrier()

    nch = jnp.where(cid == 0, _CA, _CB)

    unpack(src_a, 0)
    pltpu.make_async_copy(x_hbm.at[src_a], rows_a, sem_a).start()

    def step2(j2, _):
        j = 2 * j2
        jn = j + 1
        jnn = j + 2
        pltpu.make_async_copy(x_hbm.at[src_a], rows_a, sem_a).wait()
        fire_adds(rows_a, j, sem_a2)
        unpack(src_b, jn)

        @pl.when(j2 > 0)
        def _():
            drain_adds(rows_b, sem_b2)
        pltpu.make_async_copy(x_hbm.at[src_b], rows_b, sem_b).start()
        pltpu.make_async_copy(x_hbm.at[src_b], rows_b, sem_b).wait()
        fire_adds(rows_b, jn, sem_b2)
        drain_adds(rows_a, sem_a2)

        @pl.when(jnn < nch)
        def _():
            unpack(src_a, jnn)
            pltpu.make_async_copy(x_hbm.at[src_a], rows_a, sem_a).start()
        return 0
    lax.fori_loop(0, nch // 2, step2, 0)
    drain_adds(rows_b, sem_b2)

    plsc.subcore_barrier()

    for c in range(_RT // _CH):
        pltpu.sync_copy(acc_sh.at[pl.ds(r0 + c * _CH, _CH)], rows_a)
        pltpu.sync_copy(rows_a, out_hbm.at[cid, pl.ds(r0 + c * _CH, _CH)])


def _make_sc_agg(w):
    return pl.kernel(
        functools.partial(_sc_agg_body, w),
        out_type=jax.ShapeDtypeStruct((_NC, _NP, w), jnp.float32),
        mesh=_mesh,
        scratch_types=(
            pltpu.VMEM((_CA, _CH), jnp.int32),
            pltpu.VMEM((_CH,), jnp.int32),
            pltpu.VMEM((_CH,), jnp.int32),
            pltpu.VMEM((_CH, w), jnp.float32),
            pltpu.VMEM((_CH, w), jnp.float32),
            pltpu.SemaphoreType.DMA,
            pltpu.SemaphoreType.DMA,
            pltpu.SemaphoreType.DMA,
            pltpu.SemaphoreType.DMA,
            pltpu.VMEM_SHARED((_NP, w), jnp.float32),
        ),
    )


_sc_agg = _make_sc_agg(_D)


def _sc_deg_body(idx_hbm, zeros_hbm, out_hbm, idx_v, rows_v, sem2, acc_sh):
    cid = lax.axis_index("c")
    sid = lax.axis_index("s")
    wid = cid * _NS + sid
    r0 = sid * _RT

    pltpu.sync_copy(zeros_hbm.at[pl.ds(0, _CH)], rows_v)
    for c in range(_RT // _CH):
        pltpu.sync_copy(rows_v, acc_sh.at[pl.ds(r0 + c * _CH, _CH)])
    pltpu.sync_copy(idx_hbm.at[wid], idx_v)

    def fill(i, _):
        for k in range(_D // 16):
            rows_v[i, pl.ds(k * 16, 16)] = jnp.full((16,), 1.0, jnp.float32)
        return 0
    lax.fori_loop(0, _CH, fill, 0)
    plsc.subcore_barrier()
    nch = jnp.where(cid == 0, _CA, _CB)

    def step(j, _):
        for k in range(_CH // 16):
            pk = idx_v[j, pl.ds(k * 16, 16)]
            dstk = lax.shift_right_logical(pk, 14)
            pltpu.make_async_copy(rows_v.at[pl.ds(k * 16, 16)],
                                  acc_sh.at[dstk], sem2).start(add=True)
        pltpu.make_async_copy(zeros_hbm.at[pl.ds(0, _CH)], rows_v,
                              sem2).wait()
        return 0
    lax.fori_loop(0, nch, step, 0)

    plsc.subcore_barrier()
    for c in range(_RT // _CH):
        pltpu.sync_copy(acc_sh.at[pl.ds(r0 + c * _CH, _CH)], rows_v)
        pltpu.sync_copy(rows_v, out_hbm.at[cid, pl.ds(r0 + c * _CH, _CH)])


_sc_deg = pl.kernel(
    _sc_deg_body,
    out_type=jax.ShapeDtypeStruct((_NC, _NP, _D), jnp.float32),
    mesh=_mesh,
    scratch_types=(
        pltpu.VMEM((_CA, _CH), jnp.int32),
        pltpu.VMEM((_CH, _D), jnp.float32),
        pltpu.SemaphoreType.DMA,
        pltpu.VMEM_SHARED((_NP, _D), jnp.float32),
    ),
)


def _tc_body(act, x_ref, p0_ref, p1_ref, dg_ref, ws_ref, wn_ref, b_ref, o_ref):
    s = p0_ref[...] + p1_ref[...]
    deg = dg_ref[0, :, 0:1] + dg_ref[1, :, 0:1]
    scale = 1.0 / jnp.maximum(deg, 1.0)
    o = (jnp.dot(x_ref[...], ws_ref[...], preferred_element_type=jnp.float32)
         + jnp.dot(s, wn_ref[...], preferred_element_type=jnp.float32) * scale
         + b_ref[...])
    o_ref[...] = jnp.maximum(o, 0.0) if act else o


def _tc_layer(x, p0, p1, degp, ws, wn, b2, act):
    r = 1024
    return pl.pallas_call(
        functools.partial(_tc_body, act),
        grid=(_NP // r,),
        in_specs=[
            pl.BlockSpec((r, _D), lambda i: (i, 0)),
            pl.BlockSpec((r, _D), lambda i: (i, 0)),
            pl.BlockSpec((r, _D), lambda i: (i, 0)),
            pl.BlockSpec((2, r, 16), lambda i: (0, i, 0)),
            pl.BlockSpec((_D, _D), lambda i: (0, 0)),
            pl.BlockSpec((_D, _D), lambda i: (0, 0)),
            pl.BlockSpec((1, _D), lambda i: (0, 0)),
        ],
        out_specs=pl.BlockSpec((r, _D), lambda i: (i, 0)),
        out_shape=jax.ShapeDtypeStruct((_NP, _D), jnp.float32),
    )(x, p0, p1, degp, ws, wn, b2)


def kernel(edge_index, emb_table, W_self_0, W_neigh_0, b_0,
           W_self_1, W_neigh_1, b_1, W_self_2, W_neigh_2, b_2):
    src = edge_index[0]
    dst = edge_index[1]
    pad = _EP - _E
    pad_src = jnp.zeros((pad,), jnp.int32)
    pad_dst = _N + 16 + (jnp.arange(pad, dtype=jnp.int32) % (_NP - _N - 16))
    srcp = jnp.concatenate([src, pad_src])
    dstp = jnp.concatenate([dst, pad_dst])
    packed = jnp.bitwise_or(srcp, jnp.left_shift(dstp, 14))
    chunks = packed.reshape(_NW * _C, _CH)
    part_a = chunks[:_NS * _CA].reshape(_NS, _CA, _CH)
    part_b = chunks[_NS * _CA:].reshape(_NS, _CB, _CH)
    part_b = jnp.pad(part_b, ((0, 0), (0, _CA - _CB), (0, 0)))
    packed = jnp.concatenate([part_a, part_b], axis=0)

    x = jnp.zeros((_NP, _D), jnp.float32).at[:_N].set(emb_table)
    zeros_d = jnp.zeros((_NP, _D), jnp.float32)

    ws = (W_self_0, W_self_1, W_self_2)
    wn = (W_neigh_0, W_neigh_1, W_neigh_2)
    bs = (b_0, b_1, b_2)

    degp = _sc_deg(packed, zeros_d)[:, :, :16]
    for l in range(3):
        p = _sc_agg(x, packed, zeros_d)
        x = _tc_layer(x, p[0], p[1], degp, ws[l], wn[l],
                      bs[l].reshape(1, _D), act=(l < 2))
    return x[:_N]
